# baseline (device time: 43308 ns/iter reference)
import jax
import jax.numpy as jnp
from jax import lax
from jax.experimental import pallas as pl
from jax.experimental.pallas import tpu as pltpu

N_DEV = 4
R = 4


def kernel(x, w_mat, scale_x, scale_w):
    m_per, k = x.shape
    _, n = w_mat.shape
    n_per = n // N_DEV
    m_piece = m_per // R

    def body(x_ref, w_ref, sx_ref, sw_ref, out_ref, send_buf, recv_buf,
             send_sems, recv_sems):
        my = lax.axis_index("i")
        scale = sx_ref[0] * sw_ref[0]

        def gemm_piece(h, r):
            dst = lax.rem(my + h, N_DEV)
            wc = w_ref[:, pl.ds(dst * n_per, n_per)]
            xp = x_ref[pl.ds(r * m_piece, m_piece), :]
            acc = lax.dot(xp, wc, preferred_element_type=jnp.int32)
            y = jnp.maximum(acc.astype(jnp.float32) * scale, 0.0)
            send_buf[h - 1, r] = y.astype(jnp.bfloat16)

        def piece_rdma(h, r, peer):
            s = (h - 1) * R + r
            return pltpu.make_async_remote_copy(
                src_ref=send_buf.at[h - 1, r],
                dst_ref=recv_buf.at[h - 1, r],
                send_sem=send_sems.at[s],
                recv_sem=recv_sems.at[s],
                device_id=(peer,),
                device_id_type=pl.DeviceIdType.MESH,
            )

        def recv_piece(h, r):
            src = lax.rem(my - h + N_DEV, N_DEV)
            piece_rdma(h, r, src).wait_recv()
            out_ref[pl.ds(src * m_per + r * m_piece, m_piece), :] = (
                recv_buf[h - 1, r].astype(jnp.float32)
            )

        gemm_piece(1, 0)

        barrier_sem = pltpu.get_barrier_semaphore()
        for h in range(1, N_DEV):
            peer = lax.rem(my + h, N_DEV)
            pl.semaphore_signal(
                barrier_sem, inc=1,
                device_id=(peer,), device_id_type=pl.DeviceIdType.MESH,
            )
        pl.semaphore_wait(barrier_sem, N_DEV - 1)

        sends = []
        for h in range(1, N_DEV):
            dst = lax.rem(my + h, N_DEV)
            for r in range(R):
                if (h, r) != (1, 0):
                    gemm_piece(h, r)
                rdma = piece_rdma(h, r, dst)
                rdma.start()
                sends.append(rdma)

        for r in range(R):
            recv_piece(1, r)

        wc = w_ref[:, pl.ds(my * n_per, n_per)]
        acc = lax.dot(x_ref[...], wc, preferred_element_type=jnp.int32)
        out_ref[pl.ds(my * m_per, m_per), :] = jnp.maximum(
            acc.astype(jnp.float32) * scale, 0.0
        )

        for h in range(2, N_DEV):
            for r in range(R):
                recv_piece(h, r)

        for rdma in sends:
            rdma.wait_send()

    return pl.pallas_call(
        body,
        out_shape=jax.ShapeDtypeStruct((N_DEV * m_per, n_per), jnp.float32),
        in_specs=[
            pl.BlockSpec(memory_space=pltpu.VMEM),
            pl.BlockSpec(memory_space=pltpu.VMEM),
            pl.BlockSpec(memory_space=pltpu.SMEM),
            pl.BlockSpec(memory_space=pltpu.SMEM),
        ],
        out_specs=pl.BlockSpec(memory_space=pltpu.VMEM),
        scratch_shapes=[
            pltpu.VMEM((N_DEV - 1, R, m_piece, n_per), jnp.bfloat16),
            pltpu.VMEM((N_DEV - 1, R, m_piece, n_per), jnp.bfloat16),
            pltpu.SemaphoreType.DMA(((N_DEV - 1) * R,)),
            pltpu.SemaphoreType.DMA(((N_DEV - 1) * R,)),
        ],
        compiler_params=pltpu.CompilerParams(collective_id=0),
    )(x, w_mat, scale_x, scale_w)


# device time: 39965 ns/iter; 1.0836x vs baseline; 1.0836x over previous
import jax
import jax.numpy as jnp
from jax import lax
from jax.experimental import pallas as pl
from jax.experimental.pallas import tpu as pltpu

N_DEV = 4
R = 4


def kernel(x, w_mat, scale_x, scale_w):
    m_per, k = x.shape
    _, n = w_mat.shape
    n_per = n // N_DEV
    m_piece = m_per // R

    def body(x_ref, w_ref, sx_ref, sw_ref, out_ref, send_buf, recv_buf,
             send_sems, recv_sems):
        my = lax.axis_index("i")
        scale = sx_ref[0] * sw_ref[0]

        def gemm_piece(h, r):
            dst = lax.rem(my + h, N_DEV)
            wc = w_ref[:, pl.ds(dst * n_per, n_per)]
            xp = x_ref[pl.ds(r * m_piece, m_piece), :]
            acc = lax.dot(xp, wc, preferred_element_type=jnp.int32)
            y = jnp.maximum(acc.astype(jnp.float32) * scale, 0.0)
            send_buf[h - 1, r] = y.astype(jnp.bfloat16)

        def piece_rdma(h, r, peer):
            s = (h - 1) * R + r
            return pltpu.make_async_remote_copy(
                src_ref=send_buf.at[h - 1, r],
                dst_ref=recv_buf.at[h - 1, r],
                send_sem=send_sems.at[s],
                recv_sem=recv_sems.at[s],
                device_id=(peer,),
                device_id_type=pl.DeviceIdType.MESH,
            )

        def recv_piece(h, r):
            src = lax.rem(my - h + N_DEV, N_DEV)
            piece_rdma(h, r, src).wait_recv()
            out_ref[pl.ds(src * m_per + r * m_piece, m_piece), :] = (
                recv_buf[h - 1, r].astype(jnp.float32)
            )

        gemm_piece(1, 0)

        barrier_sem = pltpu.get_barrier_semaphore()
        for h in range(1, N_DEV):
            peer = lax.rem(my + h, N_DEV)
            pl.semaphore_signal(
                barrier_sem, inc=1,
                device_id=(peer,), device_id_type=pl.DeviceIdType.MESH,
            )
        pl.semaphore_wait(barrier_sem, N_DEV - 1)

        sends = []
        for r in range(R):
            for h in range(1, N_DEV):
                dst = lax.rem(my + h, N_DEV)
                if (h, r) != (1, 0):
                    gemm_piece(h, r)
                rdma = piece_rdma(h, r, dst)
                rdma.start()
                sends.append(rdma)

        for r in range(R):
            recv_piece(1, r)

        wc = w_ref[:, pl.ds(my * n_per, n_per)]
        acc = lax.dot(x_ref[...], wc, preferred_element_type=jnp.int32)
        out_ref[pl.ds(my * m_per, m_per), :] = jnp.maximum(
            acc.astype(jnp.float32) * scale, 0.0
        )

        for h in range(2, N_DEV):
            for r in range(R):
                recv_piece(h, r)

        for rdma in sends:
            rdma.wait_send()

    return pl.pallas_call(
        body,
        out_shape=jax.ShapeDtypeStruct((N_DEV * m_per, n_per), jnp.float32),
        in_specs=[
            pl.BlockSpec(memory_space=pltpu.VMEM),
            pl.BlockSpec(memory_space=pltpu.VMEM),
            pl.BlockSpec(memory_space=pltpu.SMEM),
            pl.BlockSpec(memory_space=pltpu.SMEM),
        ],
        out_specs=pl.BlockSpec(memory_space=pltpu.VMEM),
        scratch_shapes=[
            pltpu.VMEM((N_DEV - 1, R, m_piece, n_per), jnp.bfloat16),
            pltpu.VMEM((N_DEV - 1, R, m_piece, n_per), jnp.bfloat16),
            pltpu.SemaphoreType.DMA(((N_DEV - 1) * R,)),
            pltpu.SemaphoreType.DMA(((N_DEV - 1) * R,)),
        ],
        compiler_params=pltpu.CompilerParams(collective_id=0),
    )(x, w_mat, scale_x, scale_w)
